# Initial kernel scaffold; baseline (speedup 1.0000x reference)
import jax
import jax.numpy as jnp
from jax.experimental import pallas as pl


def kernel(x, edge_index, edge_weight, tc1_w1, tc1_b1, tc1_w2, tc1_b2, tc1_w3, tc1_b3, cheb_w0, cheb_w1, cheb_b, tc2_w1, tc2_b1, tc2_w2, tc2_b2, tc2_w3, tc2_b3, bn_gamma, bn_beta, lin_w, lin_b):
    raise NotImplementedError("write your pallas kernel here")


# speedup vs baseline: 28.9855x; 28.9855x over previous
import functools

import jax
import jax.numpy as jnp
from jax import lax
from jax.experimental import pallas as pl
from jax.experimental.pallas import tpu as pltpu
from jax.experimental.pallas import tpu_sc as plsc

_N = 10000
_NP = 10240
_E = 640000
_EP = 641536
_CH = 112
_FH = 160
_NBLK = 1024
_GRID = _NP // _NBLK

_mesh = plsc.VectorSubcoreMesh(core_axis_name="c", subcore_axis_name="s")


@functools.partial(
    pl.kernel,
    out_type=jax.ShapeDtypeStruct((2 * _NP,), jnp.float32),
    mesh=_mesh,
    scratch_types=[
        pltpu.VMEM((1, _CH), jnp.int32),
        pltpu.VMEM((_CH,), jnp.int32),
        pltpu.VMEM((_CH,), jnp.float32),
        pltpu.VMEM((_CH,), jnp.float32),
        pltpu.VMEM((_NP // 16,), jnp.float32),
        pltpu.VMEM_SHARED((_NP,), jnp.float32),
    ],
    compiler_params=pltpu.CompilerParams(needs_layout_passes=False, use_tc_tiling_on_sc=False),
)
def _deg_kernel(row_hbm, col_hbm, ew_hbm, deg_out,
                row_v, col_v, ew_v, upd_v, z_v, deg_sh):
    c = lax.axis_index("c")
    s = lax.axis_index("s")
    nslice = _NP // 16

    def zb(j, _):
        z_v[pl.ds(j * 16, 16)] = jnp.zeros((16,), jnp.float32)
        return 0
    lax.fori_loop(0, nslice // 16, zb, 0)
    pltpu.sync_copy(z_v, deg_sh.at[pl.ds(s * nslice, nslice)])
    plsc.subcore_barrier()

    per_core = _EP // 2
    per_tile = per_core // 16
    nchunks = per_tile // _CH

    def body(i, _):
        base = pl.multiple_of(c * per_core + s * per_tile + i * _CH, 8)
        pltpu.sync_copy(row_hbm.at[pl.ds(base, _CH)], row_v.at[0])
        pltpu.sync_copy(col_hbm.at[pl.ds(base, _CH)], col_v)
        pltpu.sync_copy(ew_hbm.at[pl.ds(base, _CH)], ew_v)
        for j in range(_CH // 16):
            r = row_v[0, pl.ds(j * 16, 16)]
            cc = col_v[pl.ds(j * 16, 16)]
            w = ew_v[pl.ds(j * 16, 16)]
            upd_v[pl.ds(j * 16, 16)] = jnp.where(
                r == cc, jnp.zeros((16,), jnp.float32), w)
        pltpu.sync_copy(upd_v, deg_sh.at[row_v.at[0]], add=True)
        return 0
    lax.fori_loop(0, nchunks, body, 0)
    plsc.subcore_barrier()
    pltpu.sync_copy(deg_sh.at[pl.ds(s * nslice, nslice)],
                    deg_out.at[pl.ds(c * _NP + s * nslice, nslice)])


@functools.partial(
    pl.kernel,
    out_type=jax.ShapeDtypeStruct((2 * _NP, _FH), jnp.float32),
    mesh=_mesh,
    scratch_types=[
        pltpu.VMEM((1, _CH), jnp.int32),
        pltpu.VMEM((_CH,), jnp.int32),
        pltpu.VMEM((_CH,), jnp.float32),
        pltpu.VMEM((_CH,), jnp.float32),
        pltpu.VMEM((_NP,), jnp.float32),
        pltpu.VMEM((_CH, _FH), jnp.float32),
        pltpu.VMEM_SHARED((_NP, _FH), jnp.float32),
    ],
    compiler_params=pltpu.CompilerParams(needs_layout_passes=False, use_tc_tiling_on_sc=False),
)
def _agg_kernel(row_hbm, col_hbm, ew_hbm, dis_hbm, t0a_hbm, t0b_hbm, agg_out,
                col_v, row_v, ew_v, nrm_v, dis_v, msg_v, acc_sh):
    c = lax.axis_index("c")
    s = lax.axis_index("s")

    pltpu.sync_copy(dis_hbm, dis_v)

    def zrow(e, _):
        for q in range(_FH // 16):
            msg_v[e, pl.ds(q * 16, 16)] = jnp.zeros((16,), jnp.float32)
        return 0
    lax.fori_loop(0, _CH, zrow, 0)
    nslice = _NP // 16
    for j in range(5):
        pltpu.sync_copy(msg_v, acc_sh.at[pl.ds(s * nslice + j * _CH, _CH)])
    pltpu.sync_copy(msg_v.at[pl.ds(0, 80)],
                    acc_sh.at[pl.ds(s * nslice + 5 * _CH, 80)])
    plsc.subcore_barrier()

    per_tile = _EP // 16
    nchunks = per_tile // _CH

    def body(i, _):
        base = pl.multiple_of(s * per_tile + i * _CH, 8)
        pltpu.sync_copy(row_hbm.at[pl.ds(base, _CH)], row_v)
        pltpu.sync_copy(col_hbm.at[pl.ds(base, _CH)], col_v.at[0])
        pltpu.sync_copy(ew_hbm.at[pl.ds(base, _CH)], ew_v)

        @pl.when(c == 0)
        def _():
            pltpu.sync_copy(t0a_hbm.at[row_v], msg_v)

        @pl.when(c == 1)
        def _():
            pltpu.sync_copy(t0b_hbm.at[row_v], msg_v)

        for j in range(_CH // 16):
            r = row_v[pl.ds(j * 16, 16)]
            cc = col_v[0, pl.ds(j * 16, 16)]
            w = ew_v[pl.ds(j * 16, 16)]
            dr = plsc.load_gather(dis_v, [r])
            dc = plsc.load_gather(dis_v, [cc])
            nrm_v[pl.ds(j * 16, 16)] = jnp.where(
                r == cc, jnp.zeros((16,), jnp.float32), -(dr * w * dc))

        def scale(e, _):
            wv = plsc.load_gather(nrm_v, [jnp.full((16,), e, jnp.int32)])
            for q in range(_FH // 16):
                msg_v[e, pl.ds(q * 16, 16)] = msg_v[e, pl.ds(q * 16, 16)] * wv
            return 0
        lax.fori_loop(0, _CH, scale, 0)

        pltpu.sync_copy(msg_v, acc_sh.at[col_v.at[0]], add=True)
        return 0
    lax.fori_loop(0, nchunks, body, 0)
    plsc.subcore_barrier()

    for off, sz in ((0, _CH), (112, _CH), (224, _CH), (336, _CH),
                    (448, _CH), (560, 80)):
        sl = s * nslice + off
        pltpu.sync_copy(acc_sh.at[pl.ds(sl, sz)],
                        agg_out.at[pl.ds(c * _NP + sl, sz)])


def _tc1_body(x_ref, deg_ref, w1_ref, b1_ref, w2_ref, b2_ref, w3_ref, b3_ref,
              dis_ref, t0a_ref, t0b_ref):
    deg = deg_ref[0, :] + deg_ref[1, :]
    safe = jnp.where(deg > 0, deg, jnp.ones_like(deg))
    dis_ref[...] = jnp.where(deg > 0, lax.rsqrt(safe), jnp.zeros_like(deg))
    w1 = w1_ref[...]
    w2 = w2_ref[...]
    w3 = w3_ref[...]
    b1 = b1_ref[...]
    b2 = b2_ref[...]
    b3 = b3_ref[...]
    for t in range(10):
        def conv(w, b):
            return (x_ref[:, t:t + 1] * w[0:1, :]
                    + x_ref[:, t + 1:t + 2] * w[1:2, :]
                    + x_ref[:, t + 2:t + 3] * w[2:3, :]) + b
        h = jax.nn.relu(conv(w1, b1) * jax.nn.sigmoid(conv(w2, b2))
                        + conv(w3, b3))
        lo = (t % 5) * 32
        if t < 5:
            t0a_ref[:, lo:lo + 32] = h
        else:
            t0b_ref[:, lo:lo + 32] = h


_tc1_call = pl.pallas_call(
    _tc1_body,
    grid=(_GRID,),
    in_specs=[
        pl.BlockSpec((_NBLK, 12), lambda i: (i, 0)),
        pl.BlockSpec((2, _NBLK), lambda i: (0, i)),
        pl.BlockSpec((3, 32), lambda i: (0, 0)),
        pl.BlockSpec((1, 32), lambda i: (0, 0)),
        pl.BlockSpec((3, 32), lambda i: (0, 0)),
        pl.BlockSpec((1, 32), lambda i: (0, 0)),
        pl.BlockSpec((3, 32), lambda i: (0, 0)),
        pl.BlockSpec((1, 32), lambda i: (0, 0)),
    ],
    out_specs=[
        pl.BlockSpec((_NBLK,), lambda i: (i,)),
        pl.BlockSpec((_NBLK, _FH), lambda i: (i, 0)),
        pl.BlockSpec((_NBLK, _FH), lambda i: (i, 0)),
    ],
    out_shape=[
        jax.ShapeDtypeStruct((_NP,), jnp.float32),
        jax.ShapeDtypeStruct((_NP, _FH), jnp.float32),
        jax.ShapeDtypeStruct((_NP, _FH), jnp.float32),
    ],
)


def _tc2_body(t0a_ref, t0b_ref, ag0_ref, ag1_ref, w0_ref, w1_ref, cb_ref,
              wa_ref, ba_ref, wb_ref, bb_ref, wc_ref, bc_ref,
              g_ref, be_ref, lw_ref, lb_ref, out_ref):
    W0 = w0_ref[...]
    W1 = w1_ref[...]
    cb = cb_ref[...]
    G = []
    for t in range(10):
        tr = t0a_ref if t < 5 else t0b_ref
        ar = ag0_ref if t < 5 else ag1_ref
        lo = (t % 5) * 32
        xt = tr[:, lo:lo + 32]
        at = ar[:, lo:lo + 32]
        g = (jnp.dot(xt, W0, preferred_element_type=jnp.float32)
             + jnp.dot(at, W1, preferred_element_type=jnp.float32) + cb)
        G.append(jax.nn.relu(g))
    Wa = wa_ref[...]
    Wb = wb_ref[...]
    Wc = wc_ref[...]
    ba = ba_ref[...]
    bb = bb_ref[...]
    bc = bc_ref[...]
    Hs = []
    S = jnp.zeros((_NBLK, 32), jnp.float32)
    Q = jnp.zeros((_NBLK, 32), jnp.float32)
    for t in range(8):
        def c2(W, b):
            return (jnp.dot(G[t], W[0], preferred_element_type=jnp.float32)
                    + jnp.dot(G[t + 1], W[1], preferred_element_type=jnp.float32)
                    + jnp.dot(G[t + 2], W[2], preferred_element_type=jnp.float32)
                    + b)
        h = jax.nn.relu(c2(Wa, ba) * jax.nn.sigmoid(c2(Wb, bb)) + c2(Wc, bc))
        Hs.append(h)
        S = S + h
        Q = Q + h * h
    m = jnp.sum(S, axis=1, keepdims=True) / 256.0
    q = jnp.sum(Q, axis=1, keepdims=True) / 256.0
    var = jnp.maximum(q - m * m, 0.0)
    inv = lax.rsqrt(var + 1e-5)
    gam = jnp.reshape(g_ref[...], (_NBLK, 1))
    bet = jnp.reshape(be_ref[...], (_NBLK, 1))
    acc = jnp.zeros((_NBLK, 32), jnp.float32)
    for t in range(8):
        acc = acc + jax.nn.relu((Hs[t] - m) * inv * gam + bet)
    M = acc / 8.0
    out_ref[...] = (jnp.sum(M * lw_ref[...], axis=1, keepdims=True)
                    + lb_ref[...])


_tc2_call = pl.pallas_call(
    _tc2_body,
    grid=(_GRID,),
    in_specs=[
        pl.BlockSpec((_NBLK, _FH), lambda i: (i, 0)),
        pl.BlockSpec((_NBLK, _FH), lambda i: (i, 0)),
        pl.BlockSpec((_NBLK, _FH), lambda i: (i, 0)),
        pl.BlockSpec((_NBLK, _FH), lambda i: (i + _GRID, 0)),
        pl.BlockSpec((32, 32), lambda i: (0, 0)),
        pl.BlockSpec((32, 32), lambda i: (0, 0)),
        pl.BlockSpec((1, 32), lambda i: (0, 0)),
        pl.BlockSpec((3, 32, 32), lambda i: (0, 0, 0)),
        pl.BlockSpec((1, 32), lambda i: (0, 0)),
        pl.BlockSpec((3, 32, 32), lambda i: (0, 0, 0)),
        pl.BlockSpec((1, 32), lambda i: (0, 0)),
        pl.BlockSpec((3, 32, 32), lambda i: (0, 0, 0)),
        pl.BlockSpec((1, 32), lambda i: (0, 0)),
        pl.BlockSpec((_NBLK,), lambda i: (i,)),
        pl.BlockSpec((_NBLK,), lambda i: (i,)),
        pl.BlockSpec((1, 32), lambda i: (0, 0)),
        pl.BlockSpec((1, 1), lambda i: (0, 0)),
    ],
    out_specs=pl.BlockSpec((_NBLK, 1), lambda i: (i, 0)),
    out_shape=jax.ShapeDtypeStruct((_NP, 1), jnp.float32),
)


def kernel(x, edge_index, edge_weight, tc1_w1, tc1_b1, tc1_w2, tc1_b2,
           tc1_w3, tc1_b3, cheb_w0, cheb_w1, cheb_b, tc2_w1, tc2_b1,
           tc2_w2, tc2_b2, tc2_w3, tc2_b3, bn_gamma, bn_beta, lin_w, lin_b):
    row, col = edge_index[0], edge_index[1]
    padn = _EP - _E
    pad_idx = jnp.arange(padn, dtype=jnp.int32) % _N
    rowp = jnp.concatenate([row, pad_idx])
    colp = jnp.concatenate([col, pad_idx])
    ewp = jnp.concatenate([edge_weight, jnp.zeros((padn,), jnp.float32)])

    deg2 = _deg_kernel(rowp, colp, ewp).reshape(2, _NP)

    x2 = jnp.pad(jnp.transpose(x[0, :, :, 0]), ((0, _NP - _N), (0, 0)))
    w1 = tc1_w1[:, 0, 0, :].T
    w2 = tc1_w2[:, 0, 0, :].T
    w3 = tc1_w3[:, 0, 0, :].T
    dis, t0a, t0b = _tc1_call(x2, deg2, w1, tc1_b1[None, :], w2,
                              tc1_b2[None, :], w3, tc1_b3[None, :])

    agg = _agg_kernel(rowp, colp, ewp, dis, t0a, t0b)

    wa = jnp.transpose(tc2_w1, (3, 1, 2, 0))[:, :, 0, :]
    wb = jnp.transpose(tc2_w2, (3, 1, 2, 0))[:, :, 0, :]
    wc = jnp.transpose(tc2_w3, (3, 1, 2, 0))[:, :, 0, :]
    gam = jnp.pad(bn_gamma, (0, _NP - _N))
    bet = jnp.pad(bn_beta, (0, _NP - _N))
    out = _tc2_call(t0a, t0b, agg, agg, cheb_w0, cheb_w1, cheb_b[None, :],
                    wa, tc2_b1[None, :], wb, tc2_b2[None, :],
                    wc, tc2_b3[None, :], gam, bet, lin_w, lin_b[None, :])
    return out[:_N]


# speedup vs baseline: 31.8904x; 1.1002x over previous
import functools

import jax
import jax.numpy as jnp
from jax import lax
from jax.experimental import pallas as pl
from jax.experimental.pallas import tpu as pltpu
from jax.experimental.pallas import tpu_sc as plsc

_N = 10000
_NP = 10240
_E = 640000
_CH = 80
_FH = 160
_NBLK = 1024
_GRID = _NP // _NBLK

_mesh = plsc.VectorSubcoreMesh(core_axis_name="c", subcore_axis_name="s")
_sc_params = pltpu.CompilerParams(needs_layout_passes=False,
                                  use_tc_tiling_on_sc=False)


@functools.partial(
    pl.kernel,
    out_type=[
        jax.ShapeDtypeStruct((2 * _NP,), jnp.float32),
        jax.ShapeDtypeStruct((_E,), jnp.float32),
    ],
    mesh=_mesh,
    scratch_types=[
        pltpu.VMEM((1, _CH), jnp.int32),
        pltpu.VMEM((_CH,), jnp.int32),
        pltpu.VMEM((_CH,), jnp.float32),
        pltpu.VMEM((_CH,), jnp.float32),
        pltpu.VMEM((_NP // 16,), jnp.float32),
        pltpu.VMEM_SHARED((_NP,), jnp.float32),
    ],
    compiler_params=_sc_params,
)
def _deg_kernel(row_hbm, col_hbm, ew_hbm, deg_out, ewm_out,
                row_v, col_v, ew_v, upd_v, z_v, deg_sh):
    c = lax.axis_index("c")
    s = lax.axis_index("s")
    nslice = _NP // 16

    def zb(j, _):
        z_v[pl.ds(j * 16, 16)] = jnp.zeros((16,), jnp.float32)
        return 0
    lax.fori_loop(0, nslice // 16, zb, 0)
    pltpu.sync_copy(z_v, deg_sh.at[pl.ds(s * nslice, nslice)])
    plsc.subcore_barrier()

    per_core = _E // 2
    per_tile = per_core // 16
    nchunks = per_tile // _CH

    def body(i, _):
        base = pl.multiple_of(c * per_core + s * per_tile + i * _CH, 8)
        pltpu.sync_copy(row_hbm.at[pl.ds(base, _CH)], row_v.at[0])
        pltpu.sync_copy(col_hbm.at[pl.ds(base, _CH)], col_v)
        pltpu.sync_copy(ew_hbm.at[pl.ds(base, _CH)], ew_v)
        for j in range(_CH // 16):
            r = row_v[0, pl.ds(j * 16, 16)]
            cc = col_v[pl.ds(j * 16, 16)]
            w = ew_v[pl.ds(j * 16, 16)]
            upd_v[pl.ds(j * 16, 16)] = jnp.where(
                r == cc, jnp.zeros((16,), jnp.float32), w)
        pltpu.sync_copy(upd_v, ewm_out.at[pl.ds(base, _CH)])
        pltpu.sync_copy(upd_v, deg_sh.at[row_v.at[0]], add=True)
        return 0
    lax.fori_loop(0, nchunks, body, 0)
    plsc.subcore_barrier()
    pltpu.sync_copy(deg_sh.at[pl.ds(s * nslice, nslice)],
                    deg_out.at[pl.ds(c * _NP + s * nslice, nslice)])


@functools.partial(
    pl.kernel,
    out_type=jax.ShapeDtypeStruct((2 * _NP, _FH), jnp.float32),
    mesh=_mesh,
    scratch_types=[
        pltpu.VMEM((2, _CH), jnp.int32),
        pltpu.VMEM((2 * _CH,), jnp.int32),
        pltpu.VMEM((2 * _CH,), jnp.float32),
        pltpu.VMEM((2, _CH, _FH), jnp.float32),
        pltpu.SemaphoreType.DMA((2,)),
        pltpu.SemaphoreType.DMA((2,)),
        pltpu.VMEM_SHARED((_NP, _FH), jnp.float32),
    ],
    compiler_params=_sc_params,
)
def _agg_kernel(row_hbm, col_hbm, ewm_hbm, xsa_hbm, xsb_hbm, agg_out,
                col_v, row_v, ewm_v, msg_v, sg, ss, acc_sh):
    c = lax.axis_index("c")
    s = lax.axis_index("s")
    per_tile = _E // 16
    nch = per_tile // _CH
    nslice = _NP // 16

    def zrow(e, _):
        for q in range(_FH // 16):
            msg_v[0, e, pl.ds(q * 16, 16)] = jnp.zeros((16,), jnp.float32)
        return 0
    lax.fori_loop(0, _CH, zrow, 0)
    for j in range(nslice // _CH):
        pltpu.sync_copy(msg_v.at[0], acc_sh.at[pl.ds(s * nslice + j * _CH, _CH)])
    plsc.subcore_barrier()

    def issue(idx, b):
        base = pl.multiple_of(s * per_tile + idx * _CH, 8)
        pltpu.sync_copy(row_hbm.at[pl.ds(base, _CH)],
                        row_v.at[pl.ds(b * _CH, _CH)])
        pltpu.sync_copy(col_hbm.at[pl.ds(base, _CH)], col_v.at[b])
        pltpu.sync_copy(ewm_hbm.at[pl.ds(base, _CH)],
                        ewm_v.at[pl.ds(b * _CH, _CH)])

        @pl.when(c == 0)
        def _():
            pltpu.async_copy(xsa_hbm.at[row_v.at[pl.ds(b * _CH, _CH)]],
                             msg_v.at[b], sg.at[b])

        @pl.when(c == 1)
        def _():
            pltpu.async_copy(xsb_hbm.at[row_v.at[pl.ds(b * _CH, _CH)]],
                             msg_v.at[b], sg.at[b])

    def wait_gather(b):
        @pl.when(c == 0)
        def _():
            pltpu.make_async_copy(xsa_hbm.at[row_v.at[pl.ds(b * _CH, _CH)]],
                                  msg_v.at[b], sg.at[b]).wait()

        @pl.when(c == 1)
        def _():
            pltpu.make_async_copy(xsb_hbm.at[row_v.at[pl.ds(b * _CH, _CH)]],
                                  msg_v.at[b], sg.at[b]).wait()

    def scatter(b):
        pltpu.async_copy(msg_v.at[b], acc_sh.at[col_v.at[b]], ss.at[b],
                         add=True)

    def wait_scatter(b):
        pltpu.make_async_copy(msg_v.at[b], acc_sh.at[col_v.at[b]],
                              ss.at[b]).wait()

    def scale(b):
        def sbody(e, _):
            wv = plsc.load_gather(ewm_v, [jnp.full((16,), b * _CH + e,
                                                   jnp.int32)])
            for q in range(_FH // 16):
                msg_v[b, e, pl.ds(q * 16, 16)] = (
                    msg_v[b, e, pl.ds(q * 16, 16)] * wv)
            return 0
        lax.fori_loop(0, _CH, sbody, 0)

    issue(0, 0)

    def body(g, _):
        i0 = g * 2
        @pl.when(g >= 1)
        def _():
            wait_scatter(1)
        issue(i0 + 1, 1)
        wait_gather(0)
        scale(0)
        scatter(0)
        @pl.when(i0 + 2 < nch)
        def _():
            wait_scatter(0)
            issue(i0 + 2, 0)
        wait_gather(1)
        scale(1)
        scatter(1)
        return 0
    lax.fori_loop(0, nch // 2, body, 0)
    wait_scatter(0)
    wait_scatter(1)
    plsc.subcore_barrier()

    for j in range(nslice // _CH):
        sl = s * nslice + j * _CH
        pltpu.sync_copy(acc_sh.at[pl.ds(sl, _CH)],
                        agg_out.at[pl.ds(c * _NP + sl, _CH)])


def _tc1_body(x_ref, deg_ref, w1_ref, b1_ref, w2_ref, b2_ref, w3_ref, b3_ref,
              dis_ref, t0a_ref, t0b_ref, xsa_ref, xsb_ref):
    deg = deg_ref[0, :] + deg_ref[1, :]
    safe = jnp.where(deg > 0, deg, jnp.ones_like(deg))
    dis = jnp.where(deg > 0, lax.rsqrt(safe), jnp.zeros_like(deg))
    dis_ref[...] = dis
    dis2d = jnp.reshape(dis, (_NBLK, 1))
    w1 = w1_ref[...]
    w2 = w2_ref[...]
    w3 = w3_ref[...]
    b1 = b1_ref[...]
    b2 = b2_ref[...]
    b3 = b3_ref[...]
    for t in range(10):
        def conv(w, b):
            return (x_ref[:, t:t + 1] * w[0:1, :]
                    + x_ref[:, t + 1:t + 2] * w[1:2, :]
                    + x_ref[:, t + 2:t + 3] * w[2:3, :]) + b
        h = jax.nn.relu(conv(w1, b1) * jax.nn.sigmoid(conv(w2, b2))
                        + conv(w3, b3))
        lo = (t % 5) * 32
        if t < 5:
            t0a_ref[:, lo:lo + 32] = h
            xsa_ref[:, lo:lo + 32] = dis2d * h
        else:
            t0b_ref[:, lo:lo + 32] = h
            xsb_ref[:, lo:lo + 32] = dis2d * h


_tc1_call = pl.pallas_call(
    _tc1_body,
    grid=(_GRID,),
    in_specs=[
        pl.BlockSpec((_NBLK, 12), lambda i: (i, 0)),
        pl.BlockSpec((2, _NBLK), lambda i: (0, i)),
        pl.BlockSpec((3, 32), lambda i: (0, 0)),
        pl.BlockSpec((1, 32), lambda i: (0, 0)),
        pl.BlockSpec((3, 32), lambda i: (0, 0)),
        pl.BlockSpec((1, 32), lambda i: (0, 0)),
        pl.BlockSpec((3, 32), lambda i: (0, 0)),
        pl.BlockSpec((1, 32), lambda i: (0, 0)),
    ],
    out_specs=[
        pl.BlockSpec((_NBLK,), lambda i: (i,)),
        pl.BlockSpec((_NBLK, _FH), lambda i: (i, 0)),
        pl.BlockSpec((_NBLK, _FH), lambda i: (i, 0)),
        pl.BlockSpec((_NBLK, _FH), lambda i: (i, 0)),
        pl.BlockSpec((_NBLK, _FH), lambda i: (i, 0)),
    ],
    out_shape=[
        jax.ShapeDtypeStruct((_NP,), jnp.float32),
        jax.ShapeDtypeStruct((_NP, _FH), jnp.float32),
        jax.ShapeDtypeStruct((_NP, _FH), jnp.float32),
        jax.ShapeDtypeStruct((_NP, _FH), jnp.float32),
        jax.ShapeDtypeStruct((_NP, _FH), jnp.float32),
    ],
)


def _tc2_body(t0a_ref, t0b_ref, ag0_ref, ag1_ref, dis_ref, w0_ref, w1_ref,
              cb_ref, wa_ref, ba_ref, wb_ref, bb_ref, wc_ref, bc_ref,
              g_ref, be_ref, lw_ref, lb_ref, out_ref):
    W0 = w0_ref[...]
    W1 = w1_ref[...]
    cb = cb_ref[...]
    ndis = jnp.reshape(-dis_ref[...], (_NBLK, 1))
    G = []
    for t in range(10):
        tr = t0a_ref if t < 5 else t0b_ref
        ar = ag0_ref if t < 5 else ag1_ref
        lo = (t % 5) * 32
        xt = tr[:, lo:lo + 32]
        at = ar[:, lo:lo + 32]
        g = (jnp.dot(xt, W0, preferred_element_type=jnp.float32)
             + ndis * jnp.dot(at, W1, preferred_element_type=jnp.float32)
             + cb)
        G.append(jax.nn.relu(g))
    Wa = wa_ref[...]
    Wb = wb_ref[...]
    Wc = wc_ref[...]
    ba = ba_ref[...]
    bb = bb_ref[...]
    bc = bc_ref[...]
    Hs = []
    S = jnp.zeros((_NBLK, 32), jnp.float32)
    Q = jnp.zeros((_NBLK, 32), jnp.float32)
    for t in range(8):
        def c2(W, b):
            return (jnp.dot(G[t], W[0], preferred_element_type=jnp.float32)
                    + jnp.dot(G[t + 1], W[1], preferred_element_type=jnp.float32)
                    + jnp.dot(G[t + 2], W[2], preferred_element_type=jnp.float32)
                    + b)
        h = jax.nn.relu(c2(Wa, ba) * jax.nn.sigmoid(c2(Wb, bb)) + c2(Wc, bc))
        Hs.append(h)
        S = S + h
    m = jnp.sum(S, axis=1, keepdims=True) / 256.0
    for t in range(8):
        d = Hs[t] - m
        Q = Q + d * d
    var = jnp.sum(Q, axis=1, keepdims=True) / 256.0
    inv = lax.rsqrt(var + 1e-5)
    gam = jnp.reshape(g_ref[...], (_NBLK, 1))
    bet = jnp.reshape(be_ref[...], (_NBLK, 1))
    acc = jnp.zeros((_NBLK, 32), jnp.float32)
    for t in range(8):
        acc = acc + jax.nn.relu((Hs[t] - m) * inv * gam + bet)
    M = acc / 8.0
    out_ref[...] = (jnp.sum(M * lw_ref[...], axis=1, keepdims=True)
                    + lb_ref[...])


_tc2_call = pl.pallas_call(
    _tc2_body,
    grid=(_GRID,),
    in_specs=[
        pl.BlockSpec((_NBLK, _FH), lambda i: (i, 0)),
        pl.BlockSpec((_NBLK, _FH), lambda i: (i, 0)),
        pl.BlockSpec((_NBLK, _FH), lambda i: (i, 0)),
        pl.BlockSpec((_NBLK, _FH), lambda i: (i + _GRID, 0)),
        pl.BlockSpec((_NBLK,), lambda i: (i,)),
        pl.BlockSpec((32, 32), lambda i: (0, 0)),
        pl.BlockSpec((32, 32), lambda i: (0, 0)),
        pl.BlockSpec((1, 32), lambda i: (0, 0)),
        pl.BlockSpec((3, 32, 32), lambda i: (0, 0, 0)),
        pl.BlockSpec((1, 32), lambda i: (0, 0)),
        pl.BlockSpec((3, 32, 32), lambda i: (0, 0, 0)),
        pl.BlockSpec((1, 32), lambda i: (0, 0)),
        pl.BlockSpec((3, 32, 32), lambda i: (0, 0, 0)),
        pl.BlockSpec((1, 32), lambda i: (0, 0)),
        pl.BlockSpec((_NBLK,), lambda i: (i,)),
        pl.BlockSpec((_NBLK,), lambda i: (i,)),
        pl.BlockSpec((1, 32), lambda i: (0, 0)),
        pl.BlockSpec((1, 1), lambda i: (0, 0)),
    ],
    out_specs=pl.BlockSpec((_NBLK, 1), lambda i: (i, 0)),
    out_shape=jax.ShapeDtypeStruct((_NP, 1), jnp.float32),
)


def kernel(x, edge_index, edge_weight, tc1_w1, tc1_b1, tc1_w2, tc1_b2,
           tc1_w3, tc1_b3, cheb_w0, cheb_w1, cheb_b, tc2_w1, tc2_b1,
           tc2_w2, tc2_b2, tc2_w3, tc2_b3, bn_gamma, bn_beta, lin_w, lin_b):
    row, col = edge_index[0], edge_index[1]

    deg2_flat, ewm = _deg_kernel(row, col, edge_weight)
    deg2 = deg2_flat.reshape(2, _NP)

    x2 = jnp.pad(jnp.transpose(x[0, :, :, 0]), ((0, _NP - _N), (0, 0)))
    w1 = tc1_w1[:, 0, 0, :].T
    w2 = tc1_w2[:, 0, 0, :].T
    w3 = tc1_w3[:, 0, 0, :].T
    dis, t0a, t0b, xsa, xsb = _tc1_call(x2, deg2, w1, tc1_b1[None, :], w2,
                                        tc1_b2[None, :], w3, tc1_b3[None, :])

    agg = _agg_kernel(row, col, ewm, xsa, xsb)

    wa = jnp.transpose(tc2_w1, (3, 1, 2, 0))[:, :, 0, :]
    wb = jnp.transpose(tc2_w2, (3, 1, 2, 0))[:, :, 0, :]
    wc = jnp.transpose(tc2_w3, (3, 1, 2, 0))[:, :, 0, :]
    gam = jnp.pad(bn_gamma, (0, _NP - _N))
    bet = jnp.pad(bn_beta, (0, _NP - _N))
    out = _tc2_call(t0a, t0b, agg, agg, dis, cheb_w0, cheb_w1,
                    cheb_b[None, :], wa, tc2_b1[None, :], wb, tc2_b2[None, :],
                    wc, tc2_b3[None, :], gam, bet, lin_w, lin_b[None, :])
    return out[:_N]


# speedup vs baseline: 37.9564x; 1.1902x over previous
import functools

import jax
import jax.numpy as jnp
from jax import lax
from jax.experimental import pallas as pl
from jax.experimental.pallas import tpu as pltpu
from jax.experimental.pallas import tpu_sc as plsc

_N = 10000
_NP = 10240
_E = 640000
_CH = 80
_FH = 160
_NBLK = 1024
_GRID = _NP // _NBLK

_mesh = plsc.VectorSubcoreMesh(core_axis_name="c", subcore_axis_name="s")
_sc_params = pltpu.CompilerParams(needs_layout_passes=False,
                                  use_tc_tiling_on_sc=False)


_DCH = 128
_EPD = 647168


@functools.partial(
    pl.kernel,
    out_type=jax.ShapeDtypeStruct((2 * _NP,), jnp.float32),
    mesh=_mesh,
    scratch_types=[
        pltpu.VMEM((2, _DCH), jnp.int32),
        pltpu.VMEM((2, _DCH), jnp.float32),
        pltpu.VMEM((_NP // 16,), jnp.float32),
        pltpu.SemaphoreType.DMA((2,)),
        pltpu.SemaphoreType.DMA((2,)),
        pltpu.VMEM_SHARED((_NP,), jnp.float32),
    ],
    compiler_params=_sc_params,
)
def _deg_kernel(row_hbm, ewm_hbm, deg_out,
                row_v, upd_v, z_v, slm, sd, deg_sh):
    c = lax.axis_index("c")
    s = lax.axis_index("s")
    nslice = _NP // 16

    def zb(j, _):
        z_v[pl.ds(j * 16, 16)] = jnp.zeros((16,), jnp.float32)
        return 0
    lax.fori_loop(0, nslice // 16, zb, 0)
    pltpu.sync_copy(z_v, deg_sh.at[pl.ds(s * nslice, nslice)])
    plsc.subcore_barrier()

    per_core = _EPD // 2
    per_tile = per_core // 16
    nch = per_tile // _DCH

    def issue_load(idx, b):
        base = pl.multiple_of(c * per_core + s * per_tile + idx * _DCH, 8)
        pltpu.async_copy(row_hbm.at[pl.ds(base, _DCH)], row_v.at[b],
                         slm.at[b])
        pltpu.async_copy(ewm_hbm.at[pl.ds(base, _DCH)], upd_v.at[b],
                         slm.at[b])

    def wait_load(idx, b):
        base = pl.multiple_of(c * per_core + s * per_tile + idx * _DCH, 8)
        pltpu.make_async_copy(row_hbm.at[pl.ds(base, _DCH)], row_v.at[b],
                              slm.at[b]).wait()
        pltpu.make_async_copy(ewm_hbm.at[pl.ds(base, _DCH)], upd_v.at[b],
                              slm.at[b]).wait()

    def scatter(b):
        pltpu.async_copy(upd_v.at[b], deg_sh.at[row_v.at[b]], sd.at[b],
                         add=True)

    def wait_scatter(b):
        pltpu.make_async_copy(upd_v.at[b], deg_sh.at[row_v.at[b]],
                              sd.at[b]).wait()

    issue_load(0, 0)

    def body(g, _):
        i0 = g * 2

        @pl.when(g >= 1)
        def _():
            wait_scatter(1)
        issue_load(i0 + 1, 1)
        wait_load(i0, 0)
        scatter(0)

        @pl.when(i0 + 2 < nch)
        def _():
            wait_scatter(0)
            issue_load(i0 + 2, 0)
        wait_load(i0 + 1, 1)
        scatter(1)
        return 0
    lax.fori_loop(0, nch // 2, body, 0)
    wait_scatter(0)
    wait_scatter(1)
    plsc.subcore_barrier()
    pltpu.sync_copy(deg_sh.at[pl.ds(s * nslice, nslice)],
                    deg_out.at[pl.ds(c * _NP + s * nslice, nslice)])


def _tc0_body(r_ref, c_ref, w_ref, o_ref):
    o_ref[...] = jnp.where(r_ref[...] == c_ref[...],
                           jnp.zeros_like(w_ref[...]), w_ref[...])


_tc0_call = pl.pallas_call(
    _tc0_body,
    grid=(5,),
    in_specs=[
        pl.BlockSpec((1000, 128), lambda i: (i, 0)),
        pl.BlockSpec((1000, 128), lambda i: (i, 0)),
        pl.BlockSpec((1000, 128), lambda i: (i, 0)),
    ],
    out_specs=pl.BlockSpec((1000, 128), lambda i: (i, 0)),
    out_shape=jax.ShapeDtypeStruct((5000, 128), jnp.float32),
)


@functools.partial(
    pl.kernel,
    out_type=jax.ShapeDtypeStruct((2 * _NP, _FH), jnp.float32),
    mesh=_mesh,
    scratch_types=[
        pltpu.VMEM((2, _CH), jnp.int32),
        pltpu.VMEM((2 * _CH,), jnp.int32),
        pltpu.VMEM((2 * _CH,), jnp.float32),
        pltpu.VMEM((2, _CH, _FH), jnp.float32),
        pltpu.SemaphoreType.DMA((2,)),
        pltpu.SemaphoreType.DMA((2,)),
        pltpu.VMEM_SHARED((_NP, _FH), jnp.float32),
    ],
    compiler_params=_sc_params,
)
def _agg_kernel(row_hbm, col_hbm, ewm_hbm, xsa_hbm, xsb_hbm, agg_out,
                col_v, row_v, ewm_v, msg_v, sg, ss, acc_sh):
    c = lax.axis_index("c")
    s = lax.axis_index("s")
    per_tile = _E // 16
    nch = per_tile // _CH
    nslice = _NP // 16

    def zrow(e, _):
        for q in range(_FH // 16):
            msg_v[0, e, pl.ds(q * 16, 16)] = jnp.zeros((16,), jnp.float32)
        return 0
    lax.fori_loop(0, _CH, zrow, 0)
    for j in range(nslice // _CH):
        pltpu.sync_copy(msg_v.at[0], acc_sh.at[pl.ds(s * nslice + j * _CH, _CH)])
    plsc.subcore_barrier()

    def issue(idx, b):
        base = pl.multiple_of(s * per_tile + idx * _CH, 8)
        pltpu.sync_copy(row_hbm.at[pl.ds(base, _CH)],
                        row_v.at[pl.ds(b * _CH, _CH)])
        pltpu.sync_copy(col_hbm.at[pl.ds(base, _CH)], col_v.at[b])
        pltpu.sync_copy(ewm_hbm.at[pl.ds(base, _CH)],
                        ewm_v.at[pl.ds(b * _CH, _CH)])

        @pl.when(c == 0)
        def _():
            pltpu.async_copy(xsa_hbm.at[row_v.at[pl.ds(b * _CH, _CH)]],
                             msg_v.at[b], sg.at[b])

        @pl.when(c == 1)
        def _():
            pltpu.async_copy(xsb_hbm.at[row_v.at[pl.ds(b * _CH, _CH)]],
                             msg_v.at[b], sg.at[b])

    def wait_gather(b):
        @pl.when(c == 0)
        def _():
            pltpu.make_async_copy(xsa_hbm.at[row_v.at[pl.ds(b * _CH, _CH)]],
                                  msg_v.at[b], sg.at[b]).wait()

        @pl.when(c == 1)
        def _():
            pltpu.make_async_copy(xsb_hbm.at[row_v.at[pl.ds(b * _CH, _CH)]],
                                  msg_v.at[b], sg.at[b]).wait()

    def scatter(b):
        pltpu.async_copy(msg_v.at[b], acc_sh.at[col_v.at[b]], ss.at[b],
                         add=True)

    def wait_scatter(b):
        pltpu.make_async_copy(msg_v.at[b], acc_sh.at[col_v.at[b]],
                              ss.at[b]).wait()

    def scale(b):
        def sbody(g2, _):
            for u in range(4):
                e = g2 * 4 + u
                wv = plsc.load_gather(ewm_v, [jnp.full((16,), b * _CH + e,
                                                       jnp.int32)])
                for q in range(_FH // 16):
                    msg_v[b, e, pl.ds(q * 16, 16)] = (
                        msg_v[b, e, pl.ds(q * 16, 16)] * wv)
            return 0
        lax.fori_loop(0, _CH // 4, sbody, 0)

    issue(0, 0)

    def body(g, _):
        i0 = g * 2
        @pl.when(g >= 1)
        def _():
            wait_scatter(1)
        issue(i0 + 1, 1)
        wait_gather(0)
        scale(0)
        scatter(0)
        @pl.when(i0 + 2 < nch)
        def _():
            wait_scatter(0)
            issue(i0 + 2, 0)
        wait_gather(1)
        scale(1)
        scatter(1)
        return 0
    lax.fori_loop(0, nch // 2, body, 0)
    wait_scatter(0)
    wait_scatter(1)
    plsc.subcore_barrier()

    for j in range(nslice // _CH):
        sl = s * nslice + j * _CH
        pltpu.sync_copy(acc_sh.at[pl.ds(sl, _CH)],
                        agg_out.at[pl.ds(c * _NP + sl, _CH)])


def _tc1_body(x_ref, deg_ref, w1_ref, b1_ref, w2_ref, b2_ref, w3_ref, b3_ref,
              dis_ref, t0a_ref, t0b_ref, xsa_ref, xsb_ref):
    deg = deg_ref[0, :] + deg_ref[1, :]
    safe = jnp.where(deg > 0, deg, jnp.ones_like(deg))
    dis = jnp.where(deg > 0, lax.rsqrt(safe), jnp.zeros_like(deg))
    dis_ref[...] = dis
    dis2d = jnp.reshape(dis, (_NBLK, 1))
    w1 = w1_ref[...]
    w2 = w2_ref[...]
    w3 = w3_ref[...]
    b1 = b1_ref[...]
    b2 = b2_ref[...]
    b3 = b3_ref[...]
    for t in range(10):
        def conv(w, b):
            return (x_ref[:, t:t + 1] * w[0:1, :]
                    + x_ref[:, t + 1:t + 2] * w[1:2, :]
                    + x_ref[:, t + 2:t + 3] * w[2:3, :]) + b
        h = jax.nn.relu(conv(w1, b1) * jax.nn.sigmoid(conv(w2, b2))
                        + conv(w3, b3))
        lo = (t % 5) * 32
        if t < 5:
            t0a_ref[:, lo:lo + 32] = h
            xsa_ref[:, lo:lo + 32] = dis2d * h
        else:
            t0b_ref[:, lo:lo + 32] = h
            xsb_ref[:, lo:lo + 32] = dis2d * h


_tc1_call = pl.pallas_call(
    _tc1_body,
    grid=(_GRID,),
    in_specs=[
        pl.BlockSpec((_NBLK, 12), lambda i: (i, 0)),
        pl.BlockSpec((2, _NBLK), lambda i: (0, i)),
        pl.BlockSpec((3, 32), lambda i: (0, 0)),
        pl.BlockSpec((1, 32), lambda i: (0, 0)),
        pl.BlockSpec((3, 32), lambda i: (0, 0)),
        pl.BlockSpec((1, 32), lambda i: (0, 0)),
        pl.BlockSpec((3, 32), lambda i: (0, 0)),
        pl.BlockSpec((1, 32), lambda i: (0, 0)),
    ],
    out_specs=[
        pl.BlockSpec((_NBLK,), lambda i: (i,)),
        pl.BlockSpec((_NBLK, _FH), lambda i: (i, 0)),
        pl.BlockSpec((_NBLK, _FH), lambda i: (i, 0)),
        pl.BlockSpec((_NBLK, _FH), lambda i: (i, 0)),
        pl.BlockSpec((_NBLK, _FH), lambda i: (i, 0)),
    ],
    out_shape=[
        jax.ShapeDtypeStruct((_NP,), jnp.float32),
        jax.ShapeDtypeStruct((_NP, _FH), jnp.float32),
        jax.ShapeDtypeStruct((_NP, _FH), jnp.float32),
        jax.ShapeDtypeStruct((_NP, _FH), jnp.float32),
        jax.ShapeDtypeStruct((_NP, _FH), jnp.float32),
    ],
)


def _tc2_body(t0a_ref, t0b_ref, ag0_ref, ag1_ref, dis_ref, w0_ref, w1_ref,
              cb_ref, wa_ref, ba_ref, wb_ref, bb_ref, wc_ref, bc_ref,
              g_ref, be_ref, lw_ref, lb_ref, out_ref):
    W0 = w0_ref[...]
    W1 = w1_ref[...]
    cb = cb_ref[...]
    ndis = jnp.reshape(-dis_ref[...], (_NBLK, 1))
    G = []
    for t in range(10):
        tr = t0a_ref if t < 5 else t0b_ref
        ar = ag0_ref if t < 5 else ag1_ref
        lo = (t % 5) * 32
        xt = tr[:, lo:lo + 32]
        at = ar[:, lo:lo + 32]
        g = (jnp.dot(xt, W0, preferred_element_type=jnp.float32)
             + ndis * jnp.dot(at, W1, preferred_element_type=jnp.float32)
             + cb)
        G.append(jax.nn.relu(g))
    Wa = wa_ref[...]
    Wb = wb_ref[...]
    Wc = wc_ref[...]
    ba = ba_ref[...]
    bb = bb_ref[...]
    bc = bc_ref[...]
    Hs = []
    S = jnp.zeros((_NBLK, 32), jnp.float32)
    Q = jnp.zeros((_NBLK, 32), jnp.float32)
    for t in range(8):
        def c2(W, b):
            return (jnp.dot(G[t], W[0], preferred_element_type=jnp.float32)
                    + jnp.dot(G[t + 1], W[1], preferred_element_type=jnp.float32)
                    + jnp.dot(G[t + 2], W[2], preferred_element_type=jnp.float32)
                    + b)
        h = jax.nn.relu(c2(Wa, ba) * jax.nn.sigmoid(c2(Wb, bb)) + c2(Wc, bc))
        Hs.append(h)
        S = S + h
    m = jnp.sum(S, axis=1, keepdims=True) / 256.0
    for t in range(8):
        d = Hs[t] - m
        Q = Q + d * d
    var = jnp.sum(Q, axis=1, keepdims=True) / 256.0
    inv = lax.rsqrt(var + 1e-5)
    gam = jnp.reshape(g_ref[...], (_NBLK, 1))
    bet = jnp.reshape(be_ref[...], (_NBLK, 1))
    acc = jnp.zeros((_NBLK, 32), jnp.float32)
    for t in range(8):
        acc = acc + jax.nn.relu((Hs[t] - m) * inv * gam + bet)
    M = acc / 8.0
    out_ref[...] = (jnp.sum(M * lw_ref[...], axis=1, keepdims=True)
                    + lb_ref[...])


_tc2_call = pl.pallas_call(
    _tc2_body,
    grid=(_GRID,),
    in_specs=[
        pl.BlockSpec((_NBLK, _FH), lambda i: (i, 0)),
        pl.BlockSpec((_NBLK, _FH), lambda i: (i, 0)),
        pl.BlockSpec((_NBLK, _FH), lambda i: (i, 0)),
        pl.BlockSpec((_NBLK, _FH), lambda i: (i + _GRID, 0)),
        pl.BlockSpec((_NBLK,), lambda i: (i,)),
        pl.BlockSpec((32, 32), lambda i: (0, 0)),
        pl.BlockSpec((32, 32), lambda i: (0, 0)),
        pl.BlockSpec((1, 32), lambda i: (0, 0)),
        pl.BlockSpec((3, 32, 32), lambda i: (0, 0, 0)),
        pl.BlockSpec((1, 32), lambda i: (0, 0)),
        pl.BlockSpec((3, 32, 32), lambda i: (0, 0, 0)),
        pl.BlockSpec((1, 32), lambda i: (0, 0)),
        pl.BlockSpec((3, 32, 32), lambda i: (0, 0, 0)),
        pl.BlockSpec((1, 32), lambda i: (0, 0)),
        pl.BlockSpec((_NBLK,), lambda i: (i,)),
        pl.BlockSpec((_NBLK,), lambda i: (i,)),
        pl.BlockSpec((1, 32), lambda i: (0, 0)),
        pl.BlockSpec((1, 1), lambda i: (0, 0)),
    ],
    out_specs=pl.BlockSpec((_NBLK, 1), lambda i: (i, 0)),
    out_shape=jax.ShapeDtypeStruct((_NP, 1), jnp.float32),
)


def kernel(x, edge_index, edge_weight, tc1_w1, tc1_b1, tc1_w2, tc1_b2,
           tc1_w3, tc1_b3, cheb_w0, cheb_w1, cheb_b, tc2_w1, tc2_b1,
           tc2_w2, tc2_b2, tc2_w3, tc2_b3, bn_gamma, bn_beta, lin_w, lin_b):
    row, col = edge_index[0], edge_index[1]

    ewm = _tc0_call(row.reshape(5000, 128), col.reshape(5000, 128),
                    edge_weight.reshape(5000, 128)).reshape(-1)
    padn = _EPD - _E
    rowp = jnp.concatenate([row, jnp.arange(padn, dtype=jnp.int32) % _N])
    ewmp = jnp.concatenate([ewm, jnp.zeros((padn,), jnp.float32)])
    deg2 = _deg_kernel(rowp, ewmp).reshape(2, _NP)

    x2 = jnp.pad(jnp.transpose(x[0, :, :, 0]), ((0, _NP - _N), (0, 0)))
    w1 = tc1_w1[:, 0, 0, :].T
    w2 = tc1_w2[:, 0, 0, :].T
    w3 = tc1_w3[:, 0, 0, :].T
    dis, t0a, t0b, xsa, xsb = _tc1_call(x2, deg2, w1, tc1_b1[None, :], w2,
                                        tc1_b2[None, :], w3, tc1_b3[None, :])

    agg = _agg_kernel(row, col, ewm, xsa, xsb)

    wa = jnp.transpose(tc2_w1, (3, 1, 2, 0))[:, :, 0, :]
    wb = jnp.transpose(tc2_w2, (3, 1, 2, 0))[:, :, 0, :]
    wc = jnp.transpose(tc2_w3, (3, 1, 2, 0))[:, :, 0, :]
    gam = jnp.pad(bn_gamma, (0, _NP - _N))
    bet = jnp.pad(bn_beta, (0, _NP - _N))
    out = _tc2_call(t0a, t0b, agg, agg, dis, cheb_w0, cheb_w1,
                    cheb_b[None, :], wa, tc2_b1[None, :], wb, tc2_b2[None, :],
                    wc, tc2_b3[None, :], gam, bet, lin_w, lin_b[None, :])
    return out[:_N]
